# Initial kernel scaffold; baseline (speedup 1.0000x reference)
import jax
import jax.numpy as jnp
from jax.experimental import pallas as pl


def kernel(adj, x, W0, Wm, Ws):
    raise NotImplementedError("write your pallas kernel here")


# speedup vs baseline: 1.2160x; 1.2160x over previous
import jax
import jax.numpy as jnp
from jax.experimental import pallas as pl
from jax.experimental.pallas import tpu as pltpu


def _pick_bm(n):
    for bm in (400, 200, 80, 40, 16, 8):
        if n % bm == 0:
            return bm
    return n


def _h_kernel(x_ref, w0_ref, adj_ref, h_ref, xw0_ref):
    @pl.when(pl.program_id(0) == 0)
    def _():
        xw0_ref[...] = jnp.dot(
            x_ref[...], w0_ref[...], preferred_element_type=jnp.float32
        )

    h_ref[...] = jnp.tanh(
        jnp.dot(adj_ref[...], xw0_ref[...], preferred_element_type=jnp.float32)
    )


def _head_kernel(h_ref, wcat_ref, adj_ref, eps_ref, z_ref, m_ref, s_ref, hw_ref):
    latent = m_ref.shape[1]

    @pl.when(pl.program_id(0) == 0)
    def _():
        hw_ref[...] = jnp.dot(
            h_ref[...], wcat_ref[...], preferred_element_type=jnp.float32
        )

    acc = jnp.dot(adj_ref[...], hw_ref[...], preferred_element_type=jnp.float32)
    m = acc[:, :latent]
    s = jnp.maximum(acc[:, latent:], 0.0) + 0.0001
    m_ref[...] = m
    s_ref[...] = s
    z_ref[...] = eps_ref[...] * s + m


def kernel(adj, x, W0, Wm, Ws):
    n, d_in = x.shape
    hidden = W0.shape[1]
    latent = Wm.shape[1]
    bm = _pick_bm(n)
    grid = (n // bm,)

    h = pl.pallas_call(
        _h_kernel,
        grid=grid,
        in_specs=[
            pl.BlockSpec((n, d_in), lambda i: (0, 0)),
            pl.BlockSpec((d_in, hidden), lambda i: (0, 0)),
            pl.BlockSpec((bm, n), lambda i: (i, 0)),
        ],
        out_specs=pl.BlockSpec((bm, hidden), lambda i: (i, 0)),
        out_shape=jax.ShapeDtypeStruct((n, hidden), jnp.float32),
        scratch_shapes=[pltpu.VMEM((n, hidden), jnp.float32)],
        compiler_params=pltpu.CompilerParams(
            dimension_semantics=("arbitrary",),
        ),
    )(x, W0, adj)

    wcat = jnp.concatenate([Wm, Ws], axis=1)
    eps = jax.random.normal(jax.random.key(42), (n, latent), dtype=jnp.float32)

    out_sds = jax.ShapeDtypeStruct((n, latent), jnp.float32)
    z, m_q_z, std_q_z = pl.pallas_call(
        _head_kernel,
        grid=grid,
        in_specs=[
            pl.BlockSpec((n, hidden), lambda i: (0, 0)),
            pl.BlockSpec((hidden, 2 * latent), lambda i: (0, 0)),
            pl.BlockSpec((bm, n), lambda i: (i, 0)),
            pl.BlockSpec((bm, latent), lambda i: (i, 0)),
        ],
        out_specs=[
            pl.BlockSpec((bm, latent), lambda i: (i, 0)),
            pl.BlockSpec((bm, latent), lambda i: (i, 0)),
            pl.BlockSpec((bm, latent), lambda i: (i, 0)),
        ],
        out_shape=[out_sds, out_sds, out_sds],
        scratch_shapes=[pltpu.VMEM((n, 2 * latent), jnp.float32)],
        compiler_params=pltpu.CompilerParams(
            dimension_semantics=("arbitrary",),
        ),
    )(h, wcat, adj, eps)

    return (z, m_q_z, std_q_z)


# speedup vs baseline: 1.2171x; 1.0010x over previous
import jax
import jax.numpy as jnp
from jax.experimental import pallas as pl
from jax.experimental.pallas import tpu as pltpu


def _pick_bm(n):
    for bm in (400, 200, 80, 40, 16, 8):
        if n % bm == 0:
            return bm
    return n


def _h_kernel(x_ref, w0_ref, adj_ref, h_ref, xw0_ref):
    @pl.when(pl.program_id(0) == 0)
    def _():
        xw0_ref[...] = jnp.dot(
            x_ref[...], w0_ref[...], preferred_element_type=jnp.float32
        ).astype(jnp.bfloat16)

    adj_bf = adj_ref[...].astype(jnp.bfloat16)
    h_ref[...] = jnp.tanh(
        jnp.dot(adj_bf, xw0_ref[...], preferred_element_type=jnp.float32)
    )


def _head_kernel(h_ref, wcat_ref, adj_ref, eps_ref, z_ref, m_ref, s_ref, hw_ref):
    latent = m_ref.shape[1]

    @pl.when(pl.program_id(0) == 0)
    def _():
        hw_ref[...] = jnp.dot(
            h_ref[...], wcat_ref[...], preferred_element_type=jnp.float32
        ).astype(jnp.bfloat16)

    adj_bf = adj_ref[...].astype(jnp.bfloat16)
    acc = jnp.dot(adj_bf, hw_ref[...], preferred_element_type=jnp.float32)
    m = acc[:, :latent]
    s = jnp.maximum(acc[:, latent:], 0.0) + 0.0001
    m_ref[...] = m
    s_ref[...] = s
    z_ref[...] = eps_ref[...] * s + m


def kernel(adj, x, W0, Wm, Ws):
    n, d_in = x.shape
    hidden = W0.shape[1]
    latent = Wm.shape[1]
    bm = _pick_bm(n)
    grid = (n // bm,)

    h = pl.pallas_call(
        _h_kernel,
        grid=grid,
        in_specs=[
            pl.BlockSpec((n, d_in), lambda i: (0, 0)),
            pl.BlockSpec((d_in, hidden), lambda i: (0, 0)),
            pl.BlockSpec((bm, n), lambda i: (i, 0)),
        ],
        out_specs=pl.BlockSpec((bm, hidden), lambda i: (i, 0)),
        out_shape=jax.ShapeDtypeStruct((n, hidden), jnp.float32),
        scratch_shapes=[pltpu.VMEM((n, hidden), jnp.bfloat16)],
        compiler_params=pltpu.CompilerParams(
            dimension_semantics=("arbitrary",),
        ),
    )(x, W0, adj)

    wcat = jnp.concatenate([Wm, Ws], axis=1)
    eps = jax.random.normal(jax.random.key(42), (n, latent), dtype=jnp.float32)

    out_sds = jax.ShapeDtypeStruct((n, latent), jnp.float32)
    z, m_q_z, std_q_z = pl.pallas_call(
        _head_kernel,
        grid=grid,
        in_specs=[
            pl.BlockSpec((n, hidden), lambda i: (0, 0)),
            pl.BlockSpec((hidden, 2 * latent), lambda i: (0, 0)),
            pl.BlockSpec((bm, n), lambda i: (i, 0)),
            pl.BlockSpec((bm, latent), lambda i: (i, 0)),
        ],
        out_specs=[
            pl.BlockSpec((bm, latent), lambda i: (i, 0)),
            pl.BlockSpec((bm, latent), lambda i: (i, 0)),
            pl.BlockSpec((bm, latent), lambda i: (i, 0)),
        ],
        out_shape=[out_sds, out_sds, out_sds],
        scratch_shapes=[pltpu.VMEM((n, 2 * latent), jnp.bfloat16)],
        compiler_params=pltpu.CompilerParams(
            dimension_semantics=("arbitrary",),
        ),
    )(h, wcat, adj, eps)

    return (z, m_q_z, std_q_z)
